# baseline (speedup 1.0000x reference)
import functools

import jax
import jax.numpy as jnp
from jax import lax
from jax.experimental import pallas as pl
from jax.experimental.pallas import tpu as pltpu
from jax.experimental.pallas import tpu_sc as plsc

VOCAB = 1000000
DIM = 64
BATCH = 4096
SEQ = 50
PAD_IDX = 0

L = 16
NW = 32
B_PER_W = BATCH // NW
ROWS_PER_CHUNK = 2
CHUNK_IDX = ROWS_PER_CHUNK * SEQ
N_CHUNKS = B_PER_W // ROWS_PER_CHUNK


def _encoder_kernel(tokens_hbm, table_hbm, out_hbm,
                    idx_v, rows0_v, rows1_v, row0_v, out_v,
                    sem0, sem1):
    cid = lax.axis_index("c")
    sid = lax.axis_index("s")
    wid = sid * 2 + cid

    pltpu.sync_copy(tokens_hbm.at[pl.ds(wid * N_CHUNKS, N_CHUNKS)], idx_v)
    pltpu.sync_copy(table_hbm.at[pl.ds(0, 1)], row0_v)

    row0 = [row0_v[0, pl.ds(k * L, L)] for k in range(4)]
    zeros = jnp.zeros((L,), jnp.float32)

    iota = lax.iota(jnp.int32, L)
    one = jnp.ones((L,), jnp.int32)
    izero = jnp.zeros((L,), jnp.int32)

    def count_nonpad(c, r):
        base = r * SEQ
        cnt = izero
        for off in (0, L, 2 * L):
            toks = idx_v[c, pl.ds(base + off, L)]
            cnt = cnt + jnp.where(toks != PAD_IDX, one, izero)
        tail = idx_v[c, pl.ds(base + 34, L)]
        cnt = cnt + jnp.where(
            jnp.logical_and(iota >= 14, tail != PAD_IDX), one, izero)
        parts = [cnt[i] for i in range(L)]
        while len(parts) > 1:
            parts = [parts[i] + parts[i + 1] for i in range(0, len(parts), 2)]
        return parts[0]

    def compute_chunk(c, rows_v):
        for r in range(ROWS_PER_CHUNK):
            n1 = jnp.full((L,), count_nonpad(c, r), jnp.float32)
            recip = 1.0 / jnp.maximum(n1, 1.0)
            n0f = (SEQ - n1) * recip
            acc = [zeros, zeros, zeros, zeros]
            for t in range(SEQ):
                slot = r * SEQ + t
                for k in range(4):
                    acc[k] = acc[k] + rows_v[slot, pl.ds(k * L, L)]
            orow = 2 * c + r
            for k in range(4):
                out_v[orow, pl.ds(k * L, L)] = acc[k] * recip - n0f * row0[k]

    def gather(c, rows_v, sem):
        return pltpu.async_copy(table_hbm.at[idx_v.at[c]], rows_v, sem)

    gather(0, rows0_v, sem0).wait()

    def body(i, carry):
        c0 = 2 * i
        gather(c0 + 1, rows1_v, sem1)
        compute_chunk(c0, rows0_v)
        pltpu.make_async_copy(table_hbm.at[idx_v.at[c0 + 1]], rows1_v,
                              sem1).wait()

        @pl.when(c0 + 2 < N_CHUNKS)
        def _():
            gather(c0 + 2, rows0_v, sem0)
        compute_chunk(c0 + 1, rows1_v)

        @pl.when(c0 + 2 < N_CHUNKS)
        def _():
            pltpu.make_async_copy(table_hbm.at[idx_v.at[c0 + 2]], rows0_v,
                                  sem0).wait()
        return carry

    lax.fori_loop(0, N_CHUNKS // 2, body, 0)

    pltpu.sync_copy(out_v, out_hbm.at[pl.ds(wid * B_PER_W, B_PER_W)])


@jax.jit
def kernel(token_ids, table):
    tokens2 = token_ids.reshape(BATCH * SEQ // CHUNK_IDX, CHUNK_IDX)
    mesh = plsc.VectorSubcoreMesh(core_axis_name="c", subcore_axis_name="s")
    f = functools.partial(
        pl.kernel,
        mesh=mesh,
        compiler_params=pltpu.CompilerParams(use_tc_tiling_on_sc=False),
        out_type=jax.ShapeDtypeStruct((BATCH, DIM), jnp.float32),
        scratch_types=[
            pltpu.VMEM((N_CHUNKS, CHUNK_IDX), jnp.int32),
            pltpu.VMEM((CHUNK_IDX, DIM), jnp.float32),
            pltpu.VMEM((CHUNK_IDX, DIM), jnp.float32),
            pltpu.VMEM((1, DIM), jnp.float32),
            pltpu.VMEM((B_PER_W, DIM), jnp.float32),
            pltpu.SemaphoreType.DMA,
            pltpu.SemaphoreType.DMA,
        ],
    )(_encoder_kernel)
    return f(tokens2, table)
